# Initial kernel scaffold; baseline (speedup 1.0000x reference)
import jax
import jax.numpy as jnp
from jax.experimental import pallas as pl


def kernel(x, index, W1, b1, W2, b2):
    raise NotImplementedError("write your pallas kernel here")


# speedup vs baseline: 5.2711x; 5.2711x over previous
import functools

import jax
import jax.numpy as jnp
from jax import lax
from jax.experimental import pallas as pl
from jax.experimental.pallas import tpu as pltpu
from jax.experimental.pallas import tpu_sc as plsc

N_EDGES = 320000
N_NODES = 10000
D = 128

NC = 2
NS = 16

CH = 128
NCH = N_EDGES // CH
PCC = NCH // NC
BASE = PCC // NS
REM = PCC - BASE * NS
MAXCH = BASE + 1
MAXCHA = 88
NCH_PAD = 2504

ZROWS = 78
ROWS_PER_TILE = 8 * ZROWS
TAIL0 = NS * ROWS_PER_TILE
TAIL = N_NODES - TAIL0

_sc_mesh = plsc.VectorSubcoreMesh(core_axis_name="c", subcore_axis_name="s")


@functools.partial(
    pl.kernel,
    out_type=jax.ShapeDtypeStruct((NC * N_NODES, D), jnp.float32),
    mesh=_sc_mesh,
    scratch_types=[
        pltpu.VMEM((CH, D), jnp.float32),
        pltpu.VMEM((MAXCHA, CH), jnp.int32),
        pltpu.VMEM((ZROWS, D), jnp.float32),
        pltpu.VMEM_SHARED((N_NODES, D), jnp.float32),
    ],
)
def _sc_segment_sum(x_hbm, idx_hbm, out_hbm, xbuf, idxbuf, zbuf, acc):
    c = lax.axis_index("c")
    s = lax.axis_index("s")
    start = c * PCC + s * BASE + jnp.minimum(s, REM)
    n_chunks = BASE + (s < REM).astype(jnp.int32)

    start_al = pl.multiple_of((start // 8) * 8, 8)
    off = start - start_al
    pltpu.sync_copy(idx_hbm.at[pl.ds(start_al, MAXCHA)], idxbuf)

    def zero_row(r, _):
        for j in range(D // 16):
            zbuf[r, pl.ds(j * 16, 16)] = jnp.zeros((16,), jnp.float32)
        return 0
    lax.fori_loop(0, ZROWS, zero_row, 0)
    row0 = s * ROWS_PER_TILE
    for k in range(ROWS_PER_TILE // ZROWS):
        pltpu.sync_copy(zbuf, acc.at[pl.ds(row0 + k * ZROWS, ZROWS)])

    @pl.when(s == NS - 1)
    def _zero_tail():
        pltpu.sync_copy(zbuf.at[pl.ds(0, TAIL)], acc.at[pl.ds(TAIL0, TAIL)])

    plsc.subcore_barrier()

    def body(j, _):
        row = pl.multiple_of((start + j) * CH, 8)
        pltpu.sync_copy(x_hbm.at[pl.ds(row, CH)], xbuf)
        pltpu.sync_copy(xbuf, acc.at[idxbuf.at[off + j]], add=True)
        return 0
    lax.fori_loop(0, n_chunks, body, 0)
    plsc.subcore_barrier()

    out0 = c * N_NODES + row0
    pltpu.sync_copy(acc.at[pl.ds(row0, ROWS_PER_TILE)],
                    out_hbm.at[pl.ds(out0, ROWS_PER_TILE)])

    @pl.when(s == NS - 1)
    def _copy_tail():
        pltpu.sync_copy(acc.at[pl.ds(TAIL0, TAIL)],
                        out_hbm.at[pl.ds(c * N_NODES + TAIL0, TAIL)])


def _mlp_body(a_ref, b_ref, w1_ref, b1_ref, w2_ref, b2_ref, o_ref):
    ssum = a_ref[...] + b_ref[...]
    h = lax.dot_general(ssum, w1_ref[...], (((1,), (1,)), ((), ())),
                        preferred_element_type=jnp.float32)
    h = jnp.maximum(h + b1_ref[...], 0.0)
    o = lax.dot_general(h, w2_ref[...], (((1,), (1,)), ((), ())),
                        preferred_element_type=jnp.float32)
    o_ref[...] = o + b2_ref[...]


ROW_BLK = 1000
N_BLKS = N_NODES // ROW_BLK


def _mlp(partials, W1, b1, W2, b2):
    return pl.pallas_call(
        _mlp_body,
        grid=(N_BLKS,),
        in_specs=[
            pl.BlockSpec((ROW_BLK, D), lambda i: (i, 0)),
            pl.BlockSpec((ROW_BLK, D), lambda i: (i + N_BLKS, 0)),
            pl.BlockSpec((D, D), lambda i: (0, 0)),
            pl.BlockSpec((1, D), lambda i: (0, 0)),
            pl.BlockSpec((D, D), lambda i: (0, 0)),
            pl.BlockSpec((1, D), lambda i: (0, 0)),
        ],
        out_specs=pl.BlockSpec((ROW_BLK, D), lambda i: (i, 0)),
        out_shape=jax.ShapeDtypeStruct((N_NODES, D), jnp.float32),
    )(partials, partials, W1, b1.reshape(1, D), W2, b2.reshape(1, D))


def kernel(x, index, W1, b1, W2, b2):
    idx = index.astype(jnp.int32).reshape(NCH, CH)
    idx = jnp.pad(idx, ((0, NCH_PAD - NCH), (0, 0)))
    partials = _sc_segment_sum(x, idx)
    return _mlp(partials, W1, b1, W2, b2)


# speedup vs baseline: 7.5865x; 1.4393x over previous
import functools

import jax
import jax.numpy as jnp
from jax import lax
from jax.experimental import pallas as pl
from jax.experimental.pallas import tpu as pltpu
from jax.experimental.pallas import tpu_sc as plsc

N_EDGES = 320000
N_NODES = 10000
D = 128

NC = 2
NS = 16

CH = 128
NCH = N_EDGES // CH
PCC = NCH // NC
BASE = PCC // NS
REM = PCC - BASE * NS
MAXCH = BASE + 1
MAXCHA = 88
NCH_PAD = 2504

ZROWS = 78
ROWS_PER_TILE = 8 * ZROWS
TAIL0 = NS * ROWS_PER_TILE
TAIL = N_NODES - TAIL0

_sc_mesh = plsc.VectorSubcoreMesh(core_axis_name="c", subcore_axis_name="s")


@functools.partial(
    pl.kernel,
    out_type=jax.ShapeDtypeStruct((NC * N_NODES, D), jnp.float32),
    mesh=_sc_mesh,
    scratch_types=[
        pltpu.VMEM((CH, D), jnp.float32),
        pltpu.VMEM((CH, D), jnp.float32),
        pltpu.VMEM((MAXCHA, CH), jnp.int32),
        pltpu.VMEM_SHARED((N_NODES, D), jnp.float32),
        pltpu.SemaphoreType.DMA,
        pltpu.SemaphoreType.DMA,
    ],
)
def _sc_segment_sum(x_hbm, idx_hbm, out_hbm, xbuf0, xbuf1, idxbuf,
                    acc, fsem0, fsem1):
    c = lax.axis_index("c")
    s = lax.axis_index("s")
    start = c * PCC + s * BASE + jnp.minimum(s, REM)
    n_chunks = BASE + (s < REM).astype(jnp.int32)

    start_al = pl.multiple_of((start // 8) * 8, 8)
    off = start - start_al
    pltpu.sync_copy(idx_hbm.at[pl.ds(start_al, MAXCHA)], idxbuf)

    def zero_row(r, _):
        for j in range(D // 16):
            xbuf0[r, pl.ds(j * 16, 16)] = jnp.zeros((16,), jnp.float32)
        return 0
    lax.fori_loop(0, CH, zero_row, 0)
    row0 = s * ROWS_PER_TILE
    for k in range(ROWS_PER_TILE // CH):
        pltpu.sync_copy(xbuf0, acc.at[pl.ds(row0 + k * CH, CH)])
    zrem = ROWS_PER_TILE - (ROWS_PER_TILE // CH) * CH
    pltpu.sync_copy(xbuf0.at[pl.ds(0, zrem)],
                    acc.at[pl.ds(row0 + ROWS_PER_TILE - zrem, zrem)])

    @pl.when(s == NS - 1)
    def _zero_tail():
        pltpu.sync_copy(xbuf0.at[pl.ds(0, TAIL)], acc.at[pl.ds(TAIL0, TAIL)])

    plsc.subcore_barrier()

    def fetch(chunk, buf, sem):
        row = pl.multiple_of(jnp.minimum(chunk, NCH - 1) * CH, 8)
        pltpu.async_copy(x_hbm.at[pl.ds(row, CH)], buf, sem)

    def fetch_wait(buf, sem):
        pltpu.make_async_copy(x_hbm.at[pl.ds(0, CH)], buf, sem).wait()

    fetch(start, xbuf0, fsem0)
    fetch(start + 1, xbuf1, fsem1)

    def body(i, _):
        j0 = 2 * i
        fetch_wait(xbuf0, fsem0)
        pltpu.sync_copy(xbuf0, acc.at[idxbuf.at[off + j0]], add=True)
        fetch(start + j0 + 2, xbuf0, fsem0)
        fetch_wait(xbuf1, fsem1)
        pltpu.sync_copy(xbuf1, acc.at[idxbuf.at[off + j0 + 1]], add=True)
        fetch(start + j0 + 3, xbuf1, fsem1)
        return 0
    lax.fori_loop(0, BASE // 2, body, 0)

    fetch_wait(xbuf0, fsem0)
    fetch_wait(xbuf1, fsem1)

    @pl.when(s < REM)
    def _tail_chunk():
        pltpu.sync_copy(xbuf0, acc.at[idxbuf.at[off + BASE]], add=True)

    plsc.subcore_barrier()

    out0 = c * N_NODES + row0
    pltpu.sync_copy(acc.at[pl.ds(row0, ROWS_PER_TILE)],
                    out_hbm.at[pl.ds(out0, ROWS_PER_TILE)])

    @pl.when(s == NS - 1)
    def _copy_tail():
        pltpu.sync_copy(acc.at[pl.ds(TAIL0, TAIL)],
                        out_hbm.at[pl.ds(c * N_NODES + TAIL0, TAIL)])


def _mlp_body(a_ref, b_ref, w1_ref, b1_ref, w2_ref, b2_ref, o_ref):
    ssum = a_ref[...] + b_ref[...]
    h = lax.dot_general(ssum, w1_ref[...], (((1,), (1,)), ((), ())),
                        preferred_element_type=jnp.float32)
    h = jnp.maximum(h + b1_ref[...], 0.0)
    o = lax.dot_general(h, w2_ref[...], (((1,), (1,)), ((), ())),
                        preferred_element_type=jnp.float32)
    o_ref[...] = o + b2_ref[...]


ROW_BLK = 1000
N_BLKS = N_NODES // ROW_BLK


def _mlp(partials, W1, b1, W2, b2):
    return pl.pallas_call(
        _mlp_body,
        grid=(N_BLKS,),
        in_specs=[
            pl.BlockSpec((ROW_BLK, D), lambda i: (i, 0)),
            pl.BlockSpec((ROW_BLK, D), lambda i: (i + N_BLKS, 0)),
            pl.BlockSpec((D, D), lambda i: (0, 0)),
            pl.BlockSpec((1, D), lambda i: (0, 0)),
            pl.BlockSpec((D, D), lambda i: (0, 0)),
            pl.BlockSpec((1, D), lambda i: (0, 0)),
        ],
        out_specs=pl.BlockSpec((ROW_BLK, D), lambda i: (i, 0)),
        out_shape=jax.ShapeDtypeStruct((N_NODES, D), jnp.float32),
    )(partials, partials, W1, b1.reshape(1, D), W2, b2.reshape(1, D))


def kernel(x, index, W1, b1, W2, b2):
    idx = index.astype(jnp.int32).reshape(NCH, CH)
    idx = jnp.pad(idx, ((0, NCH_PAD - NCH), (0, 0)))
    partials = _sc_segment_sum(x, idx)
    return _mlp(partials, W1, b1, W2, b2)
